# baseline (speedup 1.0000x reference)
import functools

import jax
import jax.numpy as jnp
from jax import lax
from jax.experimental import pallas as pl
from jax.experimental.pallas import tpu as pltpu
from jax.experimental.pallas import tpu_sc as plsc

D = 64
DP = 128
N0 = 4096
N1 = 50
NC, NS = 2, 16
NW = NC * NS
CHUNK = 128

_mesh = plsc.VectorSubcoreMesh(core_axis_name="c", subcore_axis_name="s")


@functools.partial(
    pl.kernel,
    mesh=_mesh,
    out_type=jax.ShapeDtypeStruct((N1, D, N0), jnp.float32),
    scratch_types=[
        pltpu.VMEM((N1, CHUNK), jnp.int32),
        pltpu.VMEM((CHUNK, DP), jnp.float32),
        pltpu.VMEM((CHUNK, DP), jnp.float32),
        pltpu.VMEM((8, 8, CHUNK), jnp.float32),
        pltpu.VMEM((8, 8, CHUNK), jnp.float32),
        pltpu.SemaphoreType.DMA,
        pltpu.SemaphoreType.DMA,
        pltpu.SemaphoreType.DMA,
        pltpu.SemaphoreType.DMA,
    ],
    compiler_params=pltpu.CompilerParams(needs_layout_passes=False),
)
def _gather_kernel(idx_hbm, table_hbm, out_hbm, idx_v, buf0, buf1,
                   obuf0, obuf1, g0, g1, o0, o1):
    nc = lax.axis_index("s") * NC + lax.axis_index("c")
    pltpu.sync_copy(idx_hbm.at[:, nc], idx_v)

    lane = lax.iota(jnp.int32, 16)

    def fire(n1, buf, gsem):
        pltpu.async_copy(table_hbm.at[idx_v.at[n1]], buf, gsem)

    def wait_gather(buf, gsem):
        pltpu.make_async_copy(table_hbm.at[idx_v.at[0]], buf, gsem).wait()

    def transpose_tile(buf, obuf):
        for d in range(D):
            col = jnp.full((16,), d, jnp.int32)
            for lb in range(8):
                v = plsc.load_gather(buf, [lane + lb * 16, col])
                obuf[d // 8, d % 8, pl.ds(lb * 16, 16)] = v

    def flush_tile(n1, obuf, osem):
        for db in range(8):
            pltpu.async_copy(
                obuf.at[db],
                out_hbm.at[n1, pl.ds(db * 8, 8), pl.ds(nc * CHUNK, CHUNK)],
                osem,
            )

    def drain_flush(obuf, osem):
        for db in range(8):
            pltpu.make_async_copy(
                obuf.at[db],
                out_hbm.at[0, pl.ds(db * 8, 8), pl.ds(nc * CHUNK, CHUNK)],
                osem,
            ).wait()

    fire(0, buf0, g0)

    def body(p, carry):
        a = 2 * p
        fire(a + 1, buf1, g1)
        wait_gather(buf0, g0)

        @pl.when(p > 0)
        def _():
            drain_flush(obuf0, o0)

        transpose_tile(buf0, obuf0)
        flush_tile(a, obuf0, o0)

        @pl.when(p < N1 // 2 - 1)
        def _():
            fire(a + 2, buf0, g0)

        wait_gather(buf1, g1)

        @pl.when(p > 0)
        def _():
            drain_flush(obuf1, o1)

        transpose_tile(buf1, obuf1)
        flush_tile(a + 1, obuf1, o1)
        return carry

    lax.fori_loop(0, N1 // 2, body, 0)
    drain_flush(obuf0, o0)
    drain_flush(obuf1, o1)


def kernel(words, table):
    idx = words.T.reshape(N1, NW, CHUNK)
    tpad = jnp.pad(table, ((0, 0), (0, DP - D)))
    out3d = _gather_kernel(idx, tpad)
    return out3d.transpose(2, 0, 1)


# speedup vs baseline: 1.2704x; 1.2704x over previous
import functools

import jax
import jax.numpy as jnp
from jax import lax
from jax.experimental import pallas as pl
from jax.experimental.pallas import tpu as pltpu
from jax.experimental.pallas import tpu_sc as plsc

D = 64
DP = 128
N0 = 4096
N1 = 50
NC, NS = 2, 16
NW = NC * NS
CHUNK = 128

_mesh = plsc.VectorSubcoreMesh(core_axis_name="c", subcore_axis_name="s")


@functools.partial(
    pl.kernel,
    mesh=_mesh,
    out_type=jax.ShapeDtypeStruct((N1, D, N0), jnp.float32),
    scratch_types=[
        pltpu.VMEM((N1, CHUNK), jnp.int32),
        pltpu.VMEM((CHUNK, DP), jnp.float32),
        pltpu.VMEM((CHUNK, DP), jnp.float32),
        pltpu.VMEM((8, 8, CHUNK), jnp.float32),
        pltpu.VMEM((8, 8, CHUNK), jnp.float32),
        pltpu.SemaphoreType.DMA,
        pltpu.SemaphoreType.DMA,
        pltpu.SemaphoreType.DMA,
        pltpu.SemaphoreType.DMA,
    ],
    compiler_params=pltpu.CompilerParams(needs_layout_passes=False),
)
def _gather_kernel(idx_hbm, table_hbm, out_hbm, idx_v, buf0, buf1,
                   obuf0, obuf1, g0, g1, o0, o1):
    nc = lax.axis_index("s") * NC + lax.axis_index("c")
    pltpu.sync_copy(idx_hbm.at[:, nc], idx_v)

    lane = lax.iota(jnp.int32, 16)

    def fire(n1, buf, gsem):
        pltpu.async_copy(table_hbm.at[idx_v.at[n1]], buf, gsem)

    def wait_gather(buf, gsem):
        pltpu.make_async_copy(table_hbm.at[idx_v.at[0]], buf, gsem).wait()

    rows = [lane + lb * 16 for lb in range(8)]

    def transpose_tile(buf, obuf):
        obuf2 = obuf.reshape(D, CHUNK)

        @plsc.parallel_loop(0, D, 1, unroll=4)
        def body(d):
            col = jnp.broadcast_to(d, (16,)).astype(jnp.int32)
            for lb in range(8):
                v = plsc.load_gather(buf, [rows[lb], col])
                obuf2[d, pl.ds(lb * 16, 16)] = v

    def flush_tile(n1, obuf, osem):
        for db in range(8):
            pltpu.async_copy(
                obuf.at[db],
                out_hbm.at[n1, pl.ds(db * 8, 8), pl.ds(nc * CHUNK, CHUNK)],
                osem,
            )

    def drain_flush(obuf, osem):
        for db in range(8):
            pltpu.make_async_copy(
                obuf.at[db],
                out_hbm.at[0, pl.ds(db * 8, 8), pl.ds(nc * CHUNK, CHUNK)],
                osem,
            ).wait()

    fire(0, buf0, g0)

    def body(p, carry):
        a = 2 * p
        fire(a + 1, buf1, g1)
        wait_gather(buf0, g0)

        @pl.when(p > 0)
        def _():
            drain_flush(obuf0, o0)

        transpose_tile(buf0, obuf0)
        flush_tile(a, obuf0, o0)

        @pl.when(p < N1 // 2 - 1)
        def _():
            fire(a + 2, buf0, g0)

        wait_gather(buf1, g1)

        @pl.when(p > 0)
        def _():
            drain_flush(obuf1, o1)

        transpose_tile(buf1, obuf1)
        flush_tile(a + 1, obuf1, o1)
        return carry

    lax.fori_loop(0, N1 // 2, body, 0)
    drain_flush(obuf0, o0)
    drain_flush(obuf1, o1)


def kernel(words, table):
    idx = words.T.reshape(N1, NW, CHUNK)
    tpad = jnp.pad(table, ((0, 0), (0, DP - D)))
    out3d = _gather_kernel(idx, tpad)
    return out3d.transpose(2, 0, 1)
